# baseline (speedup 1.0000x reference)
import jax
import jax.numpy as jnp
from jax import lax
from jax.experimental import pallas as pl
from jax.experimental.pallas import tpu as pltpu

B, T, D = 2, 2048, 1024
H, DH, E, K = 16, 64, 8, 2
BT = B * T
TB = 256
TQ = 1024

_DN = (((1,), (1,)), ((), ()))
_DNR = (((1,), (0,)), ((), ()))


def _dotb(a, b, dn=_DN):
    return lax.dot_general(a.astype(jnp.bfloat16), b.astype(jnp.bfloat16),
                           dn, preferred_element_type=jnp.float32)


def _top2_mask(a):
    n = a.shape[0]
    lane = lax.broadcasted_iota(jnp.int32, (n, E * H), 1)
    e_idx = lane // H
    rank = jnp.zeros((n, E * H), dtype=jnp.float32)
    for s in range(1, E):
        sh = jnp.concatenate([a[:, s * H:], a[:, :s * H]], axis=1)
        src_e = (e_idx + (E - s)) % E
        tie = src_e < e_idx
        cmp = (sh > a) | ((sh == a) & tie)
        rank = rank + cmp.astype(jnp.float32)
    return (rank < K).astype(jnp.float32)


def _stage1(x_ref, wq_ref, wk_ref, wsp_ref, wdp_ref, wvf_ref,
            expand_ref, tile_ref, mop_ref, q_ref, k_ref, v_ref, mo_ref):
    x = x_ref[...]
    q_ref[...] = _dotb(x, wq_ref[...]).astype(jnp.bfloat16)
    k_ref[...] = _dotb(x, wk_ref[...]).astype(jnp.bfloat16)
    ev = _dotb(x, wvf_ref[...])
    gl = _dotb(x, wsp_ref[...])
    go = _dotb(x, wdp_ref[...])

    gates = (jax.nn.sigmoid(gl) * _top2_mask(gl)).astype(jnp.bfloat16)
    evr = ev.astype(jnp.bfloat16)
    expand = expand_ref[...]
    tile = tile_ref[...]
    v = jnp.zeros((x.shape[0], H * DH), dtype=jnp.float32)
    for e in range(E):
        a_e = _dotb(gates[:, e * H:(e + 1) * H], expand, _DNR)
        b_e = _dotb(evr[:, e * DH:(e + 1) * DH], tile, _DNR)
        v = v + a_e * b_e
    v_ref[...] = v.astype(jnp.bfloat16)

    mo_ref[...] = _dotb(_top2_mask(go), mop_ref[...], _DNR).astype(jnp.bfloat16)


def _attn(q_ref, k_ref, v_ref, mp_ref, wof_ref, sel_ref, red2_ref,
          res_ref, u_ref):
    hh = pl.program_id(2)
    qq = q_ref[0, :, :]
    kk = k_ref[0, :, :]
    vv = v_ref[0, :, :]
    outs = []
    for j in (0, 1):
        q = qq[:, j * DH:(j + 1) * DH] * jnp.bfloat16(DH ** -0.5)
        k = kk[:, j * DH:(j + 1) * DH]
        v = vv[:, j * DH:(j + 1) * DH]
        s = _dotb(q, k)
        p = jnp.exp(s)
        l = jnp.sum(p, axis=1, keepdims=True)
        outs.append((_dotb(p, v, _DNR) / l).astype(jnp.bfloat16))
    out2 = jnp.concatenate(outs, axis=1).astype(jnp.float32)

    mask2 = mp_ref[0, :, :].astype(jnp.float32)
    red2 = red2_ref[...]
    parts = []
    for e in range(E):
        a_e = _dotb(mask2, sel_ref[e], _DNR)
        parts.append(_dotb(a_e * out2, red2, _DNR))
    du = jnp.concatenate(parts, axis=1)

    @pl.when(hh == 0)
    def _():
        u_ref[...] = du

    @pl.when(hh != 0)
    def _():
        u_ref[...] = u_ref[...] + du

    @pl.when(hh == H // 2 - 1)
    def _():
        res_ref[0, :, :] = _dotb(u_ref[...], wof_ref[...], _DNR)


@jax.jit
def kernel(x, Wq, Wk, Ws, Wd, Wv, Wo):
    x2 = x.reshape(BT, D)
    Wsp = Ws.reshape(H, E, D).transpose(1, 0, 2).reshape(E * H, D)
    Wdp = Wd.reshape(H, E, D).transpose(1, 0, 2).reshape(E * H, D)
    Wvf = Wv.reshape(E * DH, D)
    Wof = Wo.transpose(0, 2, 1).reshape(E * DH, D)

    eyeH = jnp.eye(H, dtype=jnp.float32)
    eyeF = jnp.eye(DH, dtype=jnp.float32)
    expand = jnp.repeat(eyeH, DH, axis=1)
    tile = jnp.tile(eyeF, (1, H))
    mop = jnp.zeros((E * H, H * DH), jnp.float32)
    le = jnp.arange(E * H) // H
    lh = jnp.arange(E * H) % H
    mop = mop.at[jnp.arange(E * H), lh * DH + le].set(1.0)
    sel = jnp.zeros((E, 2 * DH, 2 * DH), jnp.float32)
    for e in range(E):
        for j in range(2):
            sel = sel.at[e, j * DH + e, j * DH:(j + 1) * DH].set(1.0)
    red2 = jnp.tile(eyeF, (2, 1))

    n1 = BT // TB
    q, k, v, mo = pl.pallas_call(
        _stage1,
        grid=(n1,),
        in_specs=[
            pl.BlockSpec((TB, D), lambda i: (i, 0)),
            pl.BlockSpec((H * DH, D), lambda i: (0, 0)),
            pl.BlockSpec((H * DH, D), lambda i: (0, 0)),
            pl.BlockSpec((E * H, D), lambda i: (0, 0)),
            pl.BlockSpec((E * H, D), lambda i: (0, 0)),
            pl.BlockSpec((E * DH, D), lambda i: (0, 0)),
            pl.BlockSpec((H, H * DH), lambda i: (0, 0)),
            pl.BlockSpec((DH, H * DH), lambda i: (0, 0)),
            pl.BlockSpec((E * H, H * DH), lambda i: (0, 0)),
        ],
        out_specs=[
            pl.BlockSpec((TB, H * DH), lambda i: (i, 0)),
            pl.BlockSpec((TB, H * DH), lambda i: (i, 0)),
            pl.BlockSpec((TB, H * DH), lambda i: (i, 0)),
            pl.BlockSpec((TB, H * DH), lambda i: (i, 0)),
        ],
        out_shape=[
            jax.ShapeDtypeStruct((BT, H * DH), jnp.bfloat16),
            jax.ShapeDtypeStruct((BT, H * DH), jnp.bfloat16),
            jax.ShapeDtypeStruct((BT, H * DH), jnp.bfloat16),
            jax.ShapeDtypeStruct((BT, H * DH), jnp.bfloat16),
        ],
    )(x2, Wq, Wk, Wsp, Wdp, Wvf, expand, tile, mop)

    q3 = q.reshape(B, T, H * DH)
    k3 = k.reshape(B, T, H * DH)
    v3 = v.reshape(B, T, H * DH)
    mo3 = mo.reshape(B, T, H * DH)

    res = pl.pallas_call(
        _attn,
        grid=(B, T // TQ, H // 2),
        in_specs=[
            pl.BlockSpec((1, TQ, 2 * DH), lambda b, i, hh: (b, i, hh)),
            pl.BlockSpec((1, T, 2 * DH), lambda b, i, hh: (b, 0, hh)),
            pl.BlockSpec((1, T, 2 * DH), lambda b, i, hh: (b, 0, hh)),
            pl.BlockSpec((1, TQ, 2 * DH), lambda b, i, hh: (b, i, hh)),
            pl.BlockSpec((E * DH, D), lambda b, i, hh: (0, 0)),
            pl.BlockSpec((E, 2 * DH, 2 * DH), lambda b, i, hh: (0, 0, 0)),
            pl.BlockSpec((2 * DH, DH), lambda b, i, hh: (0, 0)),
        ],
        out_specs=pl.BlockSpec((1, TQ, D), lambda b, i, hh: (b, i, 0)),
        out_shape=jax.ShapeDtypeStruct((B, T, D), jnp.float32),
        scratch_shapes=[pltpu.VMEM((TQ, E * DH), jnp.float32)],
    )(q3, k3, v3, mo3, Wof, sel, red2)

    return res


# speedup vs baseline: 1.2510x; 1.2510x over previous
import jax
import jax.numpy as jnp
from jax import lax
from jax.experimental import pallas as pl

B, T, D = 2, 2048, 1024
H, DH, E, K = 16, 64, 8, 2
BT = B * T
TB = 512
TQ = 1024

_DN = (((1,), (1,)), ((), ()))
_DNR = (((1,), (0,)), ((), ()))


def _dotb(a, b, dn=_DN):
    return lax.dot_general(a.astype(jnp.bfloat16), b.astype(jnp.bfloat16),
                           dn, preferred_element_type=jnp.float32)


def _top2_mask(a):
    n = a.shape[0]
    lane = lax.broadcasted_iota(jnp.int32, (n, E * H), 1)
    e_idx = lane // H
    rank = jnp.zeros((n, E * H), dtype=jnp.float32)
    for s in range(1, E):
        sh = jnp.concatenate([a[:, s * H:], a[:, :s * H]], axis=1)
        src_e = (e_idx + (E - s)) % E
        tie = src_e < e_idx
        cmp = (sh > a) | ((sh == a) & tie)
        rank = rank + cmp.astype(jnp.float32)
    return (rank < K).astype(jnp.float32)


def _stage1(x_ref, wq_ref, wk_ref, wsp_ref, wdp_ref, wvf_ref,
            expand_ref, tile_ref, q_ref, k_ref, v_ref, mo_ref):
    x = x_ref[...]
    q_ref[...] = _dotb(x, wq_ref[...]).astype(jnp.bfloat16)
    k_ref[...] = _dotb(x, wk_ref[...]).astype(jnp.bfloat16)
    ev = _dotb(x, wvf_ref[...])
    gl = _dotb(x, wsp_ref[...])
    go = _dotb(x, wdp_ref[...])

    gates = (jax.nn.sigmoid(gl) * _top2_mask(gl)).astype(jnp.bfloat16)
    evr = ev.astype(jnp.bfloat16)
    expand = expand_ref[...]
    tile = tile_ref[...]
    v = jnp.zeros((x.shape[0], H * DH), dtype=jnp.float32)
    for e in range(E):
        a_e = _dotb(gates[:, e * H:(e + 1) * H], expand, _DNR)
        b_e = _dotb(evr[:, e * DH:(e + 1) * DH], tile, _DNR)
        v = v + a_e * b_e
    v_ref[...] = v.astype(jnp.bfloat16)

    mo_ref[...] = _top2_mask(go).astype(jnp.bfloat16)


def _attn(q_ref, k_ref, v_ref, o_ref):
    qq = q_ref[0, :, :]
    kk = k_ref[0, :, :]
    vv = v_ref[0, :, :]
    outs = []
    for j in (0, 1):
        q = qq[:, j * DH:(j + 1) * DH] * jnp.bfloat16(DH ** -0.5)
        k = kk[:, j * DH:(j + 1) * DH]
        v = vv[:, j * DH:(j + 1) * DH]
        s = _dotb(q, k)
        p = jnp.exp(s)
        l = jnp.sum(p, axis=1, keepdims=True)
        outs.append((_dotb(p, v, _DNR) / l).astype(jnp.bfloat16))
    o_ref[0, :, :] = jnp.concatenate(outs, axis=1)


def _stage3(out_ref, mo_ref, wof_ref, expand_ref, reduce_ref, res_ref):
    out_flat = out_ref[...].astype(jnp.float32)
    mo = mo_ref[...].astype(jnp.float32)
    expand = expand_ref[...]
    red = reduce_ref[...]
    parts = []
    for e in range(E):
        a_e = _dotb(mo[:, e * H:(e + 1) * H], expand, _DNR)
        parts.append(_dotb(a_e * out_flat, red, _DNR))
    u = jnp.concatenate(parts, axis=1)
    res_ref[...] = _dotb(u, wof_ref[...], _DNR)


@jax.jit
def kernel(x, Wq, Wk, Ws, Wd, Wv, Wo):
    x2 = x.reshape(BT, D)
    Wsp = Ws.reshape(H, E, D).transpose(1, 0, 2).reshape(E * H, D)
    Wdp = Wd.reshape(H, E, D).transpose(1, 0, 2).reshape(E * H, D)
    Wvf = Wv.reshape(E * DH, D)
    Wof = Wo.transpose(0, 2, 1).reshape(E * DH, D)

    eyeH = jnp.eye(H, dtype=jnp.float32)
    eyeF = jnp.eye(DH, dtype=jnp.float32)
    expand = jnp.repeat(eyeH, DH, axis=1)
    tile = jnp.tile(eyeF, (1, H))
    red = jnp.tile(eyeF, (H, 1))

    n1 = BT // TB
    q, k, v, mo = pl.pallas_call(
        _stage1,
        grid=(n1,),
        in_specs=[
            pl.BlockSpec((TB, D), lambda i: (i, 0)),
            pl.BlockSpec((H * DH, D), lambda i: (0, 0)),
            pl.BlockSpec((H * DH, D), lambda i: (0, 0)),
            pl.BlockSpec((E * H, D), lambda i: (0, 0)),
            pl.BlockSpec((E * H, D), lambda i: (0, 0)),
            pl.BlockSpec((E * DH, D), lambda i: (0, 0)),
            pl.BlockSpec((H, H * DH), lambda i: (0, 0)),
            pl.BlockSpec((DH, H * DH), lambda i: (0, 0)),
        ],
        out_specs=[
            pl.BlockSpec((TB, H * DH), lambda i: (i, 0)),
            pl.BlockSpec((TB, H * DH), lambda i: (i, 0)),
            pl.BlockSpec((TB, H * DH), lambda i: (i, 0)),
            pl.BlockSpec((TB, E * H), lambda i: (i, 0)),
        ],
        out_shape=[
            jax.ShapeDtypeStruct((BT, H * DH), jnp.bfloat16),
            jax.ShapeDtypeStruct((BT, H * DH), jnp.bfloat16),
            jax.ShapeDtypeStruct((BT, H * DH), jnp.bfloat16),
            jax.ShapeDtypeStruct((BT, E * H), jnp.bfloat16),
        ],
    )(x2, Wq, Wk, Wsp, Wdp, Wvf, expand, tile)

    q3 = q.reshape(B, T, H * DH)
    k3 = k.reshape(B, T, H * DH)
    v3 = v.reshape(B, T, H * DH)

    out = pl.pallas_call(
        _attn,
        grid=(B, H // 2, T // TQ),
        in_specs=[
            pl.BlockSpec((1, TQ, 2 * DH), lambda b, hh, i: (b, i, hh)),
            pl.BlockSpec((1, T, 2 * DH), lambda b, hh, i: (b, 0, hh)),
            pl.BlockSpec((1, T, 2 * DH), lambda b, hh, i: (b, 0, hh)),
        ],
        out_specs=pl.BlockSpec((1, TQ, 2 * DH), lambda b, hh, i: (b, i, hh)),
        out_shape=jax.ShapeDtypeStruct((B, T, H * DH), jnp.bfloat16),
    )(q3, k3, v3)

    out2 = out.reshape(BT, H * DH)
    res = pl.pallas_call(
        _stage3,
        grid=(n1,),
        in_specs=[
            pl.BlockSpec((TB, H * DH), lambda i: (i, 0)),
            pl.BlockSpec((TB, E * H), lambda i: (i, 0)),
            pl.BlockSpec((E * DH, D), lambda i: (0, 0)),
            pl.BlockSpec((H, H * DH), lambda i: (0, 0)),
            pl.BlockSpec((H * DH, DH), lambda i: (0, 0)),
        ],
        out_specs=pl.BlockSpec((TB, D), lambda i: (i, 0)),
        out_shape=jax.ShapeDtypeStruct((BT, D), jnp.float32),
    )(out2, mo, Wof, expand, red)

    return res.reshape(B, T, D)


# speedup vs baseline: 1.2523x; 1.0011x over previous
import jax
import jax.numpy as jnp
from jax import lax
from jax.experimental import pallas as pl

B, T, D = 2, 2048, 1024
H, DH, E, K = 16, 64, 8, 2
BT = B * T
TB = 512
TQ = 2048

_DN = (((1,), (1,)), ((), ()))
_DNR = (((1,), (0,)), ((), ()))


def _dotb(a, b, dn=_DN):
    return lax.dot_general(a.astype(jnp.bfloat16), b.astype(jnp.bfloat16),
                           dn, preferred_element_type=jnp.float32)


def _top2_mask(a):
    n = a.shape[0]
    lane = lax.broadcasted_iota(jnp.int32, (n, E * H), 1)
    e_idx = lane // H
    rank = jnp.zeros((n, E * H), dtype=jnp.float32)
    for s in range(1, E):
        sh = jnp.concatenate([a[:, s * H:], a[:, :s * H]], axis=1)
        src_e = (e_idx + (E - s)) % E
        tie = src_e < e_idx
        cmp = (sh > a) | ((sh == a) & tie)
        rank = rank + cmp.astype(jnp.float32)
    return (rank < K).astype(jnp.float32)


def _stage1(x_ref, wq_ref, wk_ref, wsp_ref, wdp_ref, wvf_ref,
            expand_ref, tile_ref, q_ref, k_ref, v_ref, mo_ref):
    x = x_ref[...]
    q_ref[...] = _dotb(x, wq_ref[...]).astype(jnp.bfloat16)
    k_ref[...] = _dotb(x, wk_ref[...]).astype(jnp.bfloat16)
    ev = _dotb(x, wvf_ref[...])
    gl = _dotb(x, wsp_ref[...])
    go = _dotb(x, wdp_ref[...])

    gates = (jax.nn.sigmoid(gl) * _top2_mask(gl)).astype(jnp.bfloat16)
    evr = ev.astype(jnp.bfloat16)
    expand = expand_ref[...]
    tile = tile_ref[...]
    v = jnp.zeros((x.shape[0], H * DH), dtype=jnp.float32)
    for e in range(E):
        a_e = _dotb(gates[:, e * H:(e + 1) * H], expand, _DNR)
        b_e = _dotb(evr[:, e * DH:(e + 1) * DH], tile, _DNR)
        v = v + a_e * b_e
    v_ref[...] = v.astype(jnp.bfloat16)

    mo_ref[...] = _top2_mask(go).astype(jnp.bfloat16)


def _attn(q_ref, k_ref, v_ref, o_ref):
    qq = q_ref[0, :, :]
    kk = k_ref[0, :, :]
    vv = v_ref[0, :, :]
    outs = []
    for j in (0, 1):
        q = qq[:, j * DH:(j + 1) * DH] * jnp.bfloat16(DH ** -0.5)
        k = kk[:, j * DH:(j + 1) * DH]
        v = vv[:, j * DH:(j + 1) * DH]
        s = _dotb(q, k)
        p = jnp.exp(s)
        l = jnp.sum(p, axis=1, keepdims=True)
        outs.append((_dotb(p, v, _DNR) / l).astype(jnp.bfloat16))
    o_ref[0, :, :] = jnp.concatenate(outs, axis=1)


def _stage3(out_ref, mo_ref, wof_ref, expand_ref, reduce_ref, res_ref):
    out_flat = out_ref[...].astype(jnp.float32)
    mo = mo_ref[...].astype(jnp.float32)
    expand = expand_ref[...]
    red = reduce_ref[...]
    parts = []
    for e in range(E):
        a_e = _dotb(mo[:, e * H:(e + 1) * H], expand, _DNR)
        parts.append(_dotb(a_e * out_flat, red, _DNR))
    u = jnp.concatenate(parts, axis=1)
    res_ref[...] = _dotb(u, wof_ref[...], _DNR)


@jax.jit
def kernel(x, Wq, Wk, Ws, Wd, Wv, Wo):
    x2 = x.reshape(BT, D)
    Wsp = Ws.reshape(H, E, D).transpose(1, 0, 2).reshape(E * H, D)
    Wdp = Wd.reshape(H, E, D).transpose(1, 0, 2).reshape(E * H, D)
    Wvf = Wv.reshape(E * DH, D)
    Wof = Wo.transpose(0, 2, 1).reshape(E * DH, D)

    eyeH = jnp.eye(H, dtype=jnp.float32)
    eyeF = jnp.eye(DH, dtype=jnp.float32)
    expand = jnp.repeat(eyeH, DH, axis=1)
    tile = jnp.tile(eyeF, (1, H))
    red = jnp.tile(eyeF, (H, 1))

    n1 = BT // TB
    q, k, v, mo = pl.pallas_call(
        _stage1,
        grid=(n1,),
        in_specs=[
            pl.BlockSpec((TB, D), lambda i: (i, 0)),
            pl.BlockSpec((H * DH, D), lambda i: (0, 0)),
            pl.BlockSpec((H * DH, D), lambda i: (0, 0)),
            pl.BlockSpec((E * H, D), lambda i: (0, 0)),
            pl.BlockSpec((E * H, D), lambda i: (0, 0)),
            pl.BlockSpec((E * DH, D), lambda i: (0, 0)),
            pl.BlockSpec((H, H * DH), lambda i: (0, 0)),
            pl.BlockSpec((DH, H * DH), lambda i: (0, 0)),
        ],
        out_specs=[
            pl.BlockSpec((TB, H * DH), lambda i: (i, 0)),
            pl.BlockSpec((TB, H * DH), lambda i: (i, 0)),
            pl.BlockSpec((TB, H * DH), lambda i: (i, 0)),
            pl.BlockSpec((TB, E * H), lambda i: (i, 0)),
        ],
        out_shape=[
            jax.ShapeDtypeStruct((BT, H * DH), jnp.bfloat16),
            jax.ShapeDtypeStruct((BT, H * DH), jnp.bfloat16),
            jax.ShapeDtypeStruct((BT, H * DH), jnp.bfloat16),
            jax.ShapeDtypeStruct((BT, E * H), jnp.bfloat16),
        ],
    )(x2, Wq, Wk, Wsp, Wdp, Wvf, expand, tile)

    q3 = q.reshape(B, T, H * DH)
    k3 = k.reshape(B, T, H * DH)
    v3 = v.reshape(B, T, H * DH)

    out = pl.pallas_call(
        _attn,
        grid=(B, H // 2, T // TQ),
        in_specs=[
            pl.BlockSpec((1, TQ, 2 * DH), lambda b, hh, i: (b, i, hh)),
            pl.BlockSpec((1, T, 2 * DH), lambda b, hh, i: (b, 0, hh)),
            pl.BlockSpec((1, T, 2 * DH), lambda b, hh, i: (b, 0, hh)),
        ],
        out_specs=pl.BlockSpec((1, TQ, 2 * DH), lambda b, hh, i: (b, i, hh)),
        out_shape=jax.ShapeDtypeStruct((B, T, H * DH), jnp.bfloat16),
    )(q3, k3, v3)

    out2 = out.reshape(BT, H * DH)
    res = pl.pallas_call(
        _stage3,
        grid=(n1,),
        in_specs=[
            pl.BlockSpec((TB, H * DH), lambda i: (i, 0)),
            pl.BlockSpec((TB, E * H), lambda i: (i, 0)),
            pl.BlockSpec((E * DH, D), lambda i: (0, 0)),
            pl.BlockSpec((H, H * DH), lambda i: (0, 0)),
            pl.BlockSpec((H * DH, DH), lambda i: (0, 0)),
        ],
        out_specs=pl.BlockSpec((TB, D), lambda i: (i, 0)),
        out_shape=jax.ShapeDtypeStruct((BT, D), jnp.float32),
    )(out2, mo, Wof, expand, red)

    return res.reshape(B, T, D)
